# Initial kernel scaffold; baseline (speedup 1.0000x reference)
import jax
import jax.numpy as jnp
from jax.experimental import pallas as pl


def kernel(x, edge_index, W1_0, b1_0, W2_0, b2_0, W1_1, b1_1, W2_1, b2_1, W1_2, b1_2, W2_2, b2_2, W_out, b_out):
    raise NotImplementedError("write your pallas kernel here")


# speedup vs baseline: 4.9209x; 4.9209x over previous
import functools

import jax
import jax.numpy as jnp
from jax import lax
from jax.experimental import pallas as pl
from jax.experimental.pallas import tpu as pltpu
from jax.experimental.pallas import tpu_sc as plsc

NC = 2
NS = 16
NW = NC * NS
K = 80


@functools.lru_cache(maxsize=None)
def _make_agg(N, H, E):
  assert E % NW == 0
  EPW = E // NW
  assert EPW % K == 0
  CH = EPW // K
  RPT = (N // NS) // 8 * 8
  REM = N - RPT * NS
  ZK = RPT // 3
  assert ZK % 8 == 0 and ZK * 3 == RPT and REM % 8 == 0 and REM <= ZK

  mesh = plsc.VectorSubcoreMesh(core_axis_name="c", subcore_axis_name="s")

  @functools.partial(
      pl.kernel,
      mesh=mesh,
      out_type=jax.ShapeDtypeStruct((NC, N, H), jnp.float32),
      scratch_types=[
          pltpu.VMEM((K,), jnp.int32),
          pltpu.VMEM((K,), jnp.int32),
          pltpu.VMEM((K, H), jnp.float32),
          pltpu.VMEM((ZK, H), jnp.float32),
          pltpu.VMEM_SHARED((N, H), jnp.float32),
          pltpu.SemaphoreType.DMA,
      ],
  )
  def agg(h_hbm, src_hbm, dst_hbm, out_hbm, src_v, dst_v, rows_v, zbuf, acc,
          sem):
    c = lax.axis_index("c")
    s = lax.axis_index("s")
    wid = s * NC + c

    def zrow(i, carry):
      for j in range(H // 16):
        zbuf[i, pl.ds(j * 16, 16)] = jnp.zeros((16,), jnp.float32)
      return carry
    lax.fori_loop(0, ZK, zrow, 0)
    for r in range(3):
      pltpu.sync_copy(zbuf, acc.at[pl.ds(s * RPT + r * ZK, ZK)])

    @pl.when(s == NS - 1)
    def _():
      pltpu.sync_copy(zbuf.at[pl.ds(0, REM)], acc.at[pl.ds(NS * RPT, REM)])
    plsc.subcore_barrier()

    ebase = wid * EPW

    def step(t, carry):
      off = ebase + t * K
      pltpu.sync_copy(src_hbm.at[pl.ds(off, K)], src_v)
      pltpu.sync_copy(dst_hbm.at[pl.ds(off, K)], dst_v)
      pltpu.async_copy(h_hbm.at[src_v], rows_v, sem).wait()
      pltpu.sync_copy(rows_v, acc.at[dst_v], add=True)
      return carry
    lax.fori_loop(0, CH, step, 0)
    plsc.subcore_barrier()

    for r in range(3):
      base = s * RPT + r * ZK
      pltpu.sync_copy(acc.at[pl.ds(base, ZK)], zbuf)
      pltpu.sync_copy(zbuf, out_hbm.at[c, pl.ds(base, ZK)])

    @pl.when(s == NS - 1)
    def _():
      pltpu.sync_copy(acc.at[pl.ds(NS * RPT, REM)], zbuf.at[pl.ds(0, REM)])
      pltpu.sync_copy(zbuf.at[pl.ds(0, REM)], out_hbm.at[c, pl.ds(NS * RPT, REM)])

  return agg


def _mlp_body(h_ref, p0_ref, p1_ref, w1_ref, b1_ref, w2_ref, b2_ref, o_ref):
  z = h_ref[...] + p0_ref[...] + p1_ref[...]
  y = jnp.dot(z, w1_ref[...], preferred_element_type=jnp.float32) + b1_ref[...]
  y = jnp.maximum(y, 0.0)
  o = jnp.dot(y, w2_ref[...], preferred_element_type=jnp.float32) + b2_ref[...]
  o_ref[...] = jnp.maximum(o, 0.0)


def _mlp_head_body(h_ref, p0_ref, p1_ref, w1_ref, b1_ref, w2_ref, b2_ref,
                   wo_ref, bo_ref, o_ref):
  z = h_ref[...] + p0_ref[...] + p1_ref[...]
  y = jnp.dot(z, w1_ref[...], preferred_element_type=jnp.float32) + b1_ref[...]
  y = jnp.maximum(y, 0.0)
  o = jnp.dot(y, w2_ref[...], preferred_element_type=jnp.float32) + b2_ref[...]
  o = jnp.maximum(o, 0.0)
  o_ref[...] = (jnp.dot(o, wo_ref[...], preferred_element_type=jnp.float32)
                + bo_ref[...])


def _mlp(h, p0, p1, W1, b1, W2, b2):
  N, H = h.shape
  BN = 1000
  grid = (N // BN,)
  row_spec = pl.BlockSpec((BN, H), lambda i: (i, 0))
  w_spec = pl.BlockSpec((H, H), lambda i: (0, 0))
  b_spec = pl.BlockSpec((1, H), lambda i: (0, 0))
  return pl.pallas_call(
      _mlp_body,
      grid=grid,
      in_specs=[row_spec, row_spec, row_spec, w_spec, b_spec, w_spec, b_spec],
      out_specs=row_spec,
      out_shape=jax.ShapeDtypeStruct((N, H), jnp.float32),
  )(h, p0, p1, W1, b1.reshape(1, H), W2, b2.reshape(1, H))


def _mlp_head(h, p0, p1, W1, b1, W2, b2, Wo_pad, bo_pad):
  N, H = h.shape
  BN = 1000
  grid = (N // BN,)
  row_spec = pl.BlockSpec((BN, H), lambda i: (i, 0))
  w_spec = pl.BlockSpec((H, H), lambda i: (0, 0))
  b_spec = pl.BlockSpec((1, H), lambda i: (0, 0))
  return pl.pallas_call(
      _mlp_head_body,
      grid=grid,
      in_specs=[row_spec, row_spec, row_spec, w_spec, b_spec, w_spec, b_spec,
                w_spec, b_spec],
      out_specs=row_spec,
      out_shape=jax.ShapeDtypeStruct((N, H), jnp.float32),
  )(h, p0, p1, W1, b1.reshape(1, H), W2, b2.reshape(1, H), Wo_pad, bo_pad)


def kernel(x, edge_index,
           W1_0, b1_0, W2_0, b2_0,
           W1_1, b1_1, W2_1, b2_1,
           W1_2, b1_2, W2_2, b2_2,
           W_out, b_out):
  N, H = x.shape
  E = edge_index.shape[1]
  src = edge_index[0]
  dst = edge_index[1]

  agg = _make_agg(N, H, E)

  Wo_pad = jnp.pad(W_out, ((0, 0), (0, H - W_out.shape[1])))
  bo_pad = jnp.pad(b_out, (0, H - b_out.shape[0])).reshape(1, H)

  h = x
  p = agg(h, src, dst)
  h = _mlp(h, p[0], p[1], W1_0, b1_0, W2_0, b2_0)
  p = agg(h, src, dst)
  h = _mlp(h, p[0], p[1], W1_1, b1_1, W2_1, b2_1)
  p = agg(h, src, dst)
  out_pad = _mlp_head(h, p[0], p[1], W1_2, b1_2, W2_2, b2_2, Wo_pad, bo_pad)
  return out_pad[:, 0]
